# Initial kernel scaffold; baseline (speedup 1.0000x reference)
import jax
import jax.numpy as jnp
from jax.experimental import pallas as pl


def kernel(x, edge_index, edge_attr, u, batch, W1, b1, W2, b2):
    raise NotImplementedError("write your pallas kernel here")


# speedup vs baseline: 4.6697x; 4.6697x over previous
import functools

import jax
import jax.numpy as jnp
from jax import lax
from jax.experimental import pallas as pl
from jax.experimental.pallas import tpu as pltpu, tpu_sc as plsc

_N_NODES = 10000
_N_EDGES = 320000
_D_EDGE = 16
_D_FEAT = 128
_N_GRAPHS = 64

_NC = 2
_NS = 16
_NW = _NC * _NS
_EPT = _N_EDGES // _NW
_CHUNK = 128
_NFULL = _EPT // _CHUNK
_TAIL = _EPT - _NFULL * _CHUNK
_NPAD = 10240
_RPT = _NPAD // _NS


def _edge_scatter_body(src_hbm, attr_hbm, sum_out, cnt_out,
                       idx_v, idx_t, rows_v, rows_t, ones_v,
                       sum_sh, cnt_sh):
    cid = lax.axis_index("c")
    sid = lax.axis_index("s")
    w = cid * _NS + sid
    ebase = w * _EPT

    ones16 = jnp.ones((16,), jnp.float32)
    zeros16 = jnp.zeros((16,), jnp.float32)

    @pl.loop(0, _CHUNK)
    def _(r):
        ones_v[r, :] = ones16
        rows_v[r, :] = zeros16

    @pl.loop(0, _RPT // _CHUNK)
    def _(j):
        zslice = pl.ds(sid * _RPT + j * _CHUNK, _CHUNK)
        pltpu.sync_copy(rows_v, sum_sh.at[zslice, :])
        pltpu.sync_copy(rows_v, cnt_sh.at[zslice, :])

    plsc.subcore_barrier()

    @pl.loop(0, _NFULL)
    def _(i):
        base = ebase + i * _CHUNK
        pltpu.sync_copy(src_hbm.at[pl.ds(base, _CHUNK)], idx_v)
        pltpu.sync_copy(attr_hbm.at[pl.ds(base, _CHUNK), :], rows_v)
        pltpu.sync_copy(rows_v, sum_sh.at[idx_v], add=True)
        pltpu.sync_copy(ones_v, cnt_sh.at[idx_v], add=True)

    tbase = ebase + _NFULL * _CHUNK
    pltpu.sync_copy(src_hbm.at[pl.ds(tbase, _TAIL)], idx_t)
    pltpu.sync_copy(attr_hbm.at[pl.ds(tbase, _TAIL), :], rows_t)
    pltpu.sync_copy(rows_t, sum_sh.at[idx_t], add=True)
    pltpu.sync_copy(ones_v.at[pl.ds(0, _TAIL), :], cnt_sh.at[idx_t], add=True)

    plsc.subcore_barrier()

    @pl.loop(0, _RPT // _CHUNK)
    def _(j):
        dslice = pl.ds(sid * _RPT + j * _CHUNK, _CHUNK)
        pltpu.sync_copy(sum_sh.at[dslice, :], rows_v)
        pltpu.sync_copy(rows_v, sum_out.at[cid, dslice, :])
        pltpu.sync_copy(cnt_sh.at[dslice, :], rows_v)
        pltpu.sync_copy(rows_v, cnt_out.at[cid, dslice, :])


@functools.cache
def _edge_scatter():
  return pl.kernel(
    _edge_scatter_body,
    out_type=(
        jax.ShapeDtypeStruct((_NC, _NPAD, _D_EDGE), jnp.float32),
        jax.ShapeDtypeStruct((_NC, _NPAD, _D_EDGE), jnp.float32),
    ),
    mesh=plsc.VectorSubcoreMesh(core_axis_name="c", subcore_axis_name="s"),
    scratch_types=[
        pltpu.VMEM((_CHUNK,), jnp.int32),
        pltpu.VMEM((_TAIL,), jnp.int32),
        pltpu.VMEM((_CHUNK, _D_EDGE), jnp.float32),
        pltpu.VMEM((_TAIL, _D_EDGE), jnp.float32),
        pltpu.VMEM((_CHUNK, _D_EDGE), jnp.float32),
        pltpu.VMEM_SHARED((_NPAD, _D_EDGE), jnp.float32),
        pltpu.VMEM_SHARED((_NPAD, _D_EDGE), jnp.float32),
    ],
    name="edge_scatter_sc",
  )


def _pool_mlp_body(sum_ref, cnt_ref, x_ref, b_ref, u_ref,
                   w1_ref, b1_ref, w2_ref, b2_ref, o_ref):
    node_sum = sum_ref[0, 0:_N_NODES, :] + sum_ref[1, 0:_N_NODES, :]
    node_cnt = (cnt_ref[0, 0:_N_NODES, :] + cnt_ref[1, 0:_N_NODES, :])[:, 0:1]
    ue_node = node_sum / jnp.maximum(node_cnt, 1.0)

    gids = lax.broadcasted_iota(jnp.int32, (_N_NODES, _N_GRAPHS), 1)
    onehot = (b_ref[...] == gids).astype(jnp.float32)

    cat = jnp.concatenate(
        [ue_node, x_ref[...], jnp.ones((_N_NODES, 1), jnp.float32)], axis=1)
    seg = lax.dot_general(onehot, cat, (((0,), (0,)), ((), ())),
                          preferred_element_type=jnp.float32)
    cnt_g = jnp.maximum(seg[:, _D_EDGE + _D_FEAT:], 1.0)
    pooled = seg[:, : _D_EDGE + _D_FEAT] / cnt_g

    comb = jnp.concatenate([pooled, u_ref[...]], axis=1)
    h = jnp.maximum(
        lax.dot_general(comb, w1_ref[...], (((1,), (0,)), ((), ())),
                        preferred_element_type=jnp.float32) + b1_ref[...],
        0.0)
    o_ref[...] = lax.dot_general(h, w2_ref[...], (((1,), (0,)), ((), ())),
                                 preferred_element_type=jnp.float32) + b2_ref[...]


_pool_mlp = pl.pallas_call(
    _pool_mlp_body,
    out_shape=jax.ShapeDtypeStruct((_N_GRAPHS, 128), jnp.float32),
)


@jax.jit
def kernel(x, edge_index, edge_attr, u, batch, W1, b1, W2, b2):
    src = edge_index[0].astype(jnp.int32)
    sum_p, cnt_p = _edge_scatter()(src, edge_attr)
    batch2 = batch.astype(jnp.int32).reshape(_N_NODES, 1)
    return _pool_mlp(sum_p, cnt_p, x, batch2, u,
                     W1, b1.reshape(1, -1), W2, b2.reshape(1, -1))
